# baseline (speedup 1.0000x reference)
import functools

import jax
import jax.numpy as jnp
from jax import lax
from jax.experimental import pallas as pl
from jax.experimental.pallas import tpu as pltpu
from jax.experimental.pallas import tpu_sc as plsc

_DIM = 128
_EF = 16
_HID = 128
_NN = 10000
_NE = 320000

_NC = 2
_NS = 16
_NW = _NC * _NS
_EPW = _NE // _NW
_C = 40
_NCH = _EPW // _C
_NB = 2
_RPT = 624


def _sc_fused(P, E1, src, dst, z_acc, z_deg):
  mesh = plsc.VectorSubcoreMesh(core_axis_name="c", subcore_axis_name="s")

  scratch = [pltpu.VMEM((_C,), jnp.int32) for _ in range(_NB)]
  scratch += [pltpu.VMEM((_C,), jnp.int32) for _ in range(_NB)]
  scratch += [pltpu.VMEM((_C, _DIM), jnp.float32) for _ in range(_NB)]
  scratch += [pltpu.VMEM((_C * _DIM,), jnp.float32) for _ in range(_NB)]
  scratch += [pltpu.VMEM((_C,), jnp.float32)]
  scratch += [pltpu.VMEM((640,), jnp.float32)]
  scratch += [pltpu.VMEM_SHARED((_NN, _DIM), jnp.float32),
              pltpu.VMEM_SHARED((_NN,), jnp.float32)]
  scratch += [pltpu.SemaphoreType.DMA for _ in range(6 * _NB)]

  @functools.partial(
      pl.kernel,
      out_type=(jax.ShapeDtypeStruct((_NC * _NN, _DIM), jnp.float32),
                jax.ShapeDtypeStruct((_NC * _NN,), jnp.float32)),
      mesh=mesh,
      scratch_types=scratch,
  )
  def k(p_hbm, e1_hbm, src_hbm, dst_hbm, za_hbm, zd_hbm, out_hbm, deg_hbm,
        *rest):
    dstb = rest[0:_NB]
    srcb = rest[_NB:2 * _NB]
    gbuf = rest[2 * _NB:3 * _NB]
    ebuf = rest[3 * _NB:4 * _NB]
    ones_v = rest[4 * _NB]
    degb = rest[4 * _NB + 1]
    acc_sh = rest[4 * _NB + 2]
    deg_sh = rest[4 * _NB + 3]
    sems = rest[4 * _NB + 4:]
    sem_si = sems[0:_NB]
    sem_e = sems[_NB:2 * _NB]
    sem_g = sems[2 * _NB:3 * _NB]
    sem_s = sems[3 * _NB:4 * _NB]
    sem_d = sems[4 * _NB:5 * _NB]
    sem_di = sems[5 * _NB:6 * _NB]

    c = lax.axis_index("c")
    s = lax.axis_index("s")
    wid = c * _NS + s
    wbase = wid * _EPW

    @pl.when(s == 0)
    def _():
      pltpu.sync_copy(za_hbm, acc_sh)
      pltpu.sync_copy(zd_hbm, deg_sh)

    for i in range(_C // 16):
      ones_v[pl.ds(i * 16, 16)] = jnp.ones((16,), jnp.float32)

    plsc.subcore_barrier()

    def _edge_slice(ci):
      return pl.ds(wbase + ci * _C, _C)

    def _e1_slice(ci):
      return pl.ds((wbase + ci * _C) * _DIM, _C * _DIM)

    def loads(ci, b):
      pltpu.async_copy(src_hbm.at[_edge_slice(ci)], srcb[b], sem_si[b])
      pltpu.async_copy(e1_hbm.at[_e1_slice(ci)], ebuf[b], sem_e[b])

    def gather(ci, b, guard_scatter):
      if guard_scatter:
        pltpu.make_async_copy(gbuf[b], acc_sh.at[dstb[b]], sem_s[b]).wait()
        pltpu.make_async_copy(ones_v, deg_sh.at[dstb[b]], sem_d[b]).wait()
      pltpu.make_async_copy(src_hbm.at[_edge_slice(ci)], srcb[b],
                            sem_si[b]).wait()
      pltpu.async_copy(p_hbm.at[srcb[b]], gbuf[b], sem_g[b])
      pltpu.async_copy(dst_hbm.at[_edge_slice(ci)], dstb[b], sem_di[b])

    def process(ci, b):
      pltpu.make_async_copy(e1_hbm.at[_e1_slice(ci)], ebuf[b],
                            sem_e[b]).wait()
      pltpu.make_async_copy(p_hbm.at[srcb[b]], gbuf[b], sem_g[b]).wait()
      pltpu.make_async_copy(dst_hbm.at[_edge_slice(ci)], dstb[b],
                            sem_di[b]).wait()

      def row(r, carry):
        for j in range(_DIM // 16):
          sl = pl.ds(j * 16, 16)
          esl = pl.ds(r * _DIM + j * 16, 16)
          gbuf[b][r, sl] = jnp.maximum(gbuf[b][r, sl] + ebuf[b][esl], 0.0)
        return carry

      lax.fori_loop(0, _C, row, 0)
      pltpu.async_copy(gbuf[b], acc_sh.at[dstb[b]], sem_s[b], add=True)
      pltpu.async_copy(ones_v, deg_sh.at[dstb[b]], sem_d[b], add=True)

    loads(0, 0)
    loads(1, 1)
    gather(0, 0, guard_scatter=False)

    def body(i2, carry):
      k0 = i2 * _NB
      for u in range(_NB):
        ck = k0 + u
        b = u
        b1 = (u + 1) % _NB
        process(ck, b)

        @pl.when(ck + 2 < _NCH)
        def _():
          loads(ck + 2, b)

        @pl.when(ck + 1 < _NCH)
        def _():
          gather(ck + 1, b1, guard_scatter=True)

      return carry

    process(0, 0)
    loads(2, 0)
    gather(1, 1, guard_scatter=False)
    process(1, 1)
    loads(3, 1)
    gather(2, 0, guard_scatter=True)
    lax.fori_loop(1, _NCH // _NB, body, 0)

    for b in range(_NB):
      pltpu.make_async_copy(gbuf[b], acc_sh.at[dstb[b]], sem_s[b]).wait()
      pltpu.make_async_copy(ones_v, deg_sh.at[dstb[b]], sem_d[b]).wait()

    plsc.subcore_barrier()

    @pl.when(s < _NS - 1)
    def _():
      pltpu.sync_copy(acc_sh.at[pl.ds(s * _RPT, _RPT)],
                      out_hbm.at[pl.ds(c * _NN + s * _RPT, _RPT)])

    @pl.when(s == _NS - 1)
    def _():
      last = _NN - (_NS - 1) * _RPT
      pltpu.sync_copy(acc_sh.at[pl.ds((_NS - 1) * _RPT, last)],
                      out_hbm.at[pl.ds(c * _NN + (_NS - 1) * _RPT, last)])

    @pl.when(s < _NS - 1)
    def _():
      pltpu.sync_copy(deg_sh.at[pl.ds(s * _RPT, _RPT)], degb.at[pl.ds(0, _RPT)])
      pltpu.sync_copy(degb.at[pl.ds(0, _RPT)],
                      deg_hbm.at[pl.ds(c * _NN + s * _RPT, _RPT)])

    @pl.when(s == _NS - 1)
    def _():
      last = _NN - (_NS - 1) * _RPT
      pltpu.sync_copy(deg_sh.at[pl.ds((_NS - 1) * _RPT, last)], degb)
      pltpu.sync_copy(degb,
                      deg_hbm.at[pl.ds(c * _NN + (_NS - 1) * _RPT, last)])

  return k(P, E1, src, dst, z_acc, z_deg)


def _tc_node_mm(H, W, b):
  br = 1000

  def body(h_ref, w_ref, b_ref, o_ref):
    o_ref[...] = (
        jnp.dot(h_ref[...], w_ref[...], preferred_element_type=jnp.float32)
        + b_ref[...])

  return pl.pallas_call(
      body,
      grid=(_NN // br,),
      in_specs=[
          pl.BlockSpec((br, _DIM), lambda i: (i, 0)),
          pl.BlockSpec((_DIM, _HID), lambda i: (0, 0)),
          pl.BlockSpec((1, _HID), lambda i: (0, 0)),
      ],
      out_specs=pl.BlockSpec((br, _HID), lambda i: (i, 0)),
      out_shape=jax.ShapeDtypeStruct((_NN, _HID), jnp.float32),
  )(H, W, b)


def _tc_e1(e_packed, Wbd):
  br = 1000
  rows = _NE // 8
  cols = 8 * _HID

  def body(e_ref, w_ref, o_ref):
    o_ref[...] = jnp.dot(e_ref[...], w_ref[...],
                         preferred_element_type=jnp.float32)

  return pl.pallas_call(
      body,
      grid=(rows // br,),
      in_specs=[
          pl.BlockSpec((br, 8 * _EF), lambda i: (i, 0)),
          pl.BlockSpec((8 * _EF, cols), lambda i: (0, 0)),
      ],
      out_specs=pl.BlockSpec((br, cols), lambda i: (i, 0)),
      out_shape=jax.ShapeDtypeStruct((rows, cols), jnp.float32),
  )(e_packed, Wbd)


def _tc_final(H, r0, r1, d0, d1, scale, W2, b2, sW1, sb1, sW2, sb2):
  br = 1000

  def body(scale_ref, h_ref, r0_ref, r1_ref, d0_ref, d1_ref, w2_ref, b2_ref,
           w1_ref, b1_ref, sw2_ref, sb2_ref, o_ref):
    rsum = r0_ref[...] + r1_ref[...]
    deg = d0_ref[...] + d1_ref[...]
    x = (scale_ref[0] * h_ref[...]
         + jnp.dot(rsum, w2_ref[...], preferred_element_type=jnp.float32)
         + deg * b2_ref[...])
    h2 = jnp.maximum(
        jnp.dot(x, w1_ref[...], preferred_element_type=jnp.float32)
        + b1_ref[...], 0.0)
    o_ref[...] = (
        jnp.dot(h2, sw2_ref[...], preferred_element_type=jnp.float32)
        + sb2_ref[...])

  return pl.pallas_call(
      body,
      grid=(_NN // br,),
      in_specs=[
          pl.BlockSpec(memory_space=pltpu.SMEM),
          pl.BlockSpec((br, _DIM), lambda i: (i, 0)),
          pl.BlockSpec((br, _HID), lambda i: (i, 0)),
          pl.BlockSpec((br, _HID), lambda i: (i, 0)),
          pl.BlockSpec((br, 1), lambda i: (i, 0)),
          pl.BlockSpec((br, 1), lambda i: (i, 0)),
          pl.BlockSpec((_HID, _DIM), lambda i: (0, 0)),
          pl.BlockSpec((1, _DIM), lambda i: (0, 0)),
          pl.BlockSpec((_DIM, _HID), lambda i: (0, 0)),
          pl.BlockSpec((1, _HID), lambda i: (0, 0)),
          pl.BlockSpec((_HID, _DIM), lambda i: (0, 0)),
          pl.BlockSpec((1, _DIM), lambda i: (0, 0)),
      ],
      out_specs=pl.BlockSpec((br, _DIM), lambda i: (i, 0)),
      out_shape=jax.ShapeDtypeStruct((_NN, _DIM), jnp.float32),
  )(scale, H, r0, r1, d0, d1, W2, b2, sW1, sb1, sW2, sb2)


def kernel(H, edge_index, edge_attr, eps, msg_W1, msg_b1, msg_W2, msg_b2,
           self_W1, self_b1, self_W2, self_b2):
  src = edge_index[0].astype(jnp.int32)
  dst = edge_index[1].astype(jnp.int32)
  W1h = msg_W1[:_DIM]
  W1e = msg_W1[_DIM:]

  P = _tc_node_mm(H, W1h, msg_b1.reshape(1, _HID))
  Wbd = jnp.kron(jnp.eye(8, dtype=jnp.float32), W1e)
  E1 = _tc_e1(edge_attr.reshape(_NE // 8, 8 * _EF), Wbd).reshape(_NE * _HID)
  R, deg = _sc_fused(P, E1, src, dst,
                     jnp.zeros((_NN, _DIM), jnp.float32),
                     jnp.zeros((_NN,), jnp.float32))
  R = R.reshape(_NC, _NN, _HID)
  deg = deg.reshape(_NC, _NN, 1)
  scale = (1.0 + eps).astype(jnp.float32)
  return _tc_final(H, R[0], R[1], deg[0], deg[1], scale, msg_W2,
                   msg_b2.reshape(1, _DIM), self_W1, self_b1.reshape(1, _HID),
                   self_W2, self_b2.reshape(1, _DIM))


# speedup vs baseline: 1.3367x; 1.3367x over previous
import functools

import jax
import jax.numpy as jnp
from jax import lax
from jax.experimental import pallas as pl
from jax.experimental.pallas import tpu as pltpu
from jax.experimental.pallas import tpu_sc as plsc

_DIM = 128
_EF = 16
_HID = 128
_NN = 10000
_NE = 320000

_NC = 2
_NS = 16
_NW = _NC * _NS
_EPW = _NE // _NW
_C = 40
_NCH = _EPW // _C
_NB = 2
_RPT = 624


def _sc_fused(P, E1, src, dst, z_acc, z_deg):
  mesh = plsc.VectorSubcoreMesh(core_axis_name="c", subcore_axis_name="s")

  scratch = [pltpu.VMEM((_C,), jnp.int32) for _ in range(_NB)]
  scratch += [pltpu.VMEM((_C,), jnp.int32) for _ in range(_NB)]
  scratch += [pltpu.VMEM((_C, _DIM), jnp.float32) for _ in range(_NB)]
  scratch += [pltpu.VMEM((_C, _DIM), jnp.float32) for _ in range(_NB)]
  scratch += [pltpu.VMEM((_C,), jnp.float32)]
  scratch += [pltpu.VMEM((640,), jnp.float32)]
  scratch += [pltpu.VMEM_SHARED((_NN, _DIM), jnp.float32),
              pltpu.VMEM_SHARED((_NN,), jnp.float32)]
  scratch += [pltpu.SemaphoreType.DMA for _ in range(6 * _NB)]

  @functools.partial(
      pl.kernel,
      out_type=(jax.ShapeDtypeStruct((_NC * _NN, _DIM), jnp.float32),
                jax.ShapeDtypeStruct((_NC * _NN,), jnp.float32)),
      mesh=mesh,
      scratch_types=scratch,
  )
  def k(p_hbm, e1_hbm, src_hbm, dst_hbm, za_hbm, zd_hbm, out_hbm, deg_hbm,
        *rest):
    dstb = rest[0:_NB]
    srcb = rest[_NB:2 * _NB]
    gbuf = rest[2 * _NB:3 * _NB]
    ebuf = rest[3 * _NB:4 * _NB]
    ones_v = rest[4 * _NB]
    degb = rest[4 * _NB + 1]
    acc_sh = rest[4 * _NB + 2]
    deg_sh = rest[4 * _NB + 3]
    sems = rest[4 * _NB + 4:]
    sem_si = sems[0:_NB]
    sem_e = sems[_NB:2 * _NB]
    sem_g = sems[2 * _NB:3 * _NB]
    sem_s = sems[3 * _NB:4 * _NB]
    sem_d = sems[4 * _NB:5 * _NB]
    sem_di = sems[5 * _NB:6 * _NB]

    c = lax.axis_index("c")
    s = lax.axis_index("s")
    wid = c * _NS + s
    wbase = wid * _EPW

    @pl.when(s == 0)
    def _():
      pltpu.sync_copy(za_hbm, acc_sh)
      pltpu.sync_copy(zd_hbm, deg_sh)

    for i in range(_C // 16):
      ones_v[pl.ds(i * 16, 16)] = jnp.ones((16,), jnp.float32)

    plsc.subcore_barrier()

    def _edge_slice(ci):
      return pl.ds(wbase + ci * _C, _C)

    def _e1_slice(ci):
      return pl.ds(wbase + ci * _C, _C)

    def loads(ci, b):
      pltpu.async_copy(src_hbm.at[_edge_slice(ci)], srcb[b], sem_si[b])
      pltpu.async_copy(e1_hbm.at[_e1_slice(ci)], ebuf[b], sem_e[b])

    def gather(ci, b, guard_scatter):
      if guard_scatter:
        pltpu.make_async_copy(gbuf[b], acc_sh.at[dstb[b]], sem_s[b]).wait()
        pltpu.make_async_copy(ones_v, deg_sh.at[dstb[b]], sem_d[b]).wait()
      pltpu.make_async_copy(src_hbm.at[_edge_slice(ci)], srcb[b],
                            sem_si[b]).wait()
      pltpu.async_copy(p_hbm.at[srcb[b]], gbuf[b], sem_g[b])
      pltpu.async_copy(dst_hbm.at[_edge_slice(ci)], dstb[b], sem_di[b])

    def process(ci, b):
      pltpu.make_async_copy(e1_hbm.at[_e1_slice(ci)], ebuf[b],
                            sem_e[b]).wait()
      pltpu.make_async_copy(p_hbm.at[srcb[b]], gbuf[b], sem_g[b]).wait()
      pltpu.make_async_copy(dst_hbm.at[_edge_slice(ci)], dstb[b],
                            sem_di[b]).wait()

      def row(r, carry):
        for j in range(_DIM // 16):
          sl = pl.ds(j * 16, 16)
          gbuf[b][r, sl] = jnp.maximum(gbuf[b][r, sl] + ebuf[b][r, sl], 0.0)
        return carry

      lax.fori_loop(0, _C, row, 0)
      pltpu.async_copy(gbuf[b], acc_sh.at[dstb[b]], sem_s[b], add=True)
      pltpu.async_copy(ones_v, deg_sh.at[dstb[b]], sem_d[b], add=True)

    loads(0, 0)
    loads(1, 1)
    gather(0, 0, guard_scatter=False)

    def body(i2, carry):
      k0 = i2 * _NB
      for u in range(_NB):
        ck = k0 + u
        b = u
        b1 = (u + 1) % _NB
        process(ck, b)

        @pl.when(ck + 2 < _NCH)
        def _():
          loads(ck + 2, b)

        @pl.when(ck + 1 < _NCH)
        def _():
          gather(ck + 1, b1, guard_scatter=True)

      return carry

    process(0, 0)
    loads(2, 0)
    gather(1, 1, guard_scatter=False)
    process(1, 1)
    loads(3, 1)
    gather(2, 0, guard_scatter=True)
    lax.fori_loop(1, _NCH // _NB, body, 0)

    for b in range(_NB):
      pltpu.make_async_copy(gbuf[b], acc_sh.at[dstb[b]], sem_s[b]).wait()
      pltpu.make_async_copy(ones_v, deg_sh.at[dstb[b]], sem_d[b]).wait()

    plsc.subcore_barrier()

    @pl.when(s < _NS - 1)
    def _():
      pltpu.sync_copy(acc_sh.at[pl.ds(s * _RPT, _RPT)],
                      out_hbm.at[pl.ds(c * _NN + s * _RPT, _RPT)])

    @pl.when(s == _NS - 1)
    def _():
      last = _NN - (_NS - 1) * _RPT
      pltpu.sync_copy(acc_sh.at[pl.ds((_NS - 1) * _RPT, last)],
                      out_hbm.at[pl.ds(c * _NN + (_NS - 1) * _RPT, last)])

    @pl.when(s < _NS - 1)
    def _():
      pltpu.sync_copy(deg_sh.at[pl.ds(s * _RPT, _RPT)], degb.at[pl.ds(0, _RPT)])
      pltpu.sync_copy(degb.at[pl.ds(0, _RPT)],
                      deg_hbm.at[pl.ds(c * _NN + s * _RPT, _RPT)])

    @pl.when(s == _NS - 1)
    def _():
      last = _NN - (_NS - 1) * _RPT
      pltpu.sync_copy(deg_sh.at[pl.ds((_NS - 1) * _RPT, last)], degb)
      pltpu.sync_copy(degb,
                      deg_hbm.at[pl.ds(c * _NN + (_NS - 1) * _RPT, last)])

  return k(P, E1, src, dst, z_acc, z_deg)


def _tc_node_mm(H, W, b):
  br = 1000

  def body(h_ref, w_ref, b_ref, o_ref):
    o_ref[...] = (
        jnp.dot(h_ref[...], w_ref[...], preferred_element_type=jnp.float32)
        + b_ref[...])

  return pl.pallas_call(
      body,
      grid=(_NN // br,),
      in_specs=[
          pl.BlockSpec((br, _DIM), lambda i: (i, 0)),
          pl.BlockSpec((_DIM, _HID), lambda i: (0, 0)),
          pl.BlockSpec((1, _HID), lambda i: (0, 0)),
      ],
      out_specs=pl.BlockSpec((br, _HID), lambda i: (i, 0)),
      out_shape=jax.ShapeDtypeStruct((_NN, _HID), jnp.float32),
  )(H, W, b)


def _tc_e1(eT, W1e):
  br = 3200

  def body(et_ref, w_ref, o_ref):
    o_ref[...] = lax.dot_general(
        et_ref[...], w_ref[...], (((0,), (0,)), ((), ())),
        preferred_element_type=jnp.float32)

  return pl.pallas_call(
      body,
      grid=(_NE // br,),
      in_specs=[
          pl.BlockSpec((_EF, br), lambda i: (0, i)),
          pl.BlockSpec((_EF, _HID), lambda i: (0, 0)),
      ],
      out_specs=pl.BlockSpec((br, _HID), lambda i: (i, 0)),
      out_shape=jax.ShapeDtypeStruct((_NE, _HID), jnp.float32),
  )(eT, W1e)


def _tc_final(H, r0, r1, d0, d1, scale, W2, b2, sW1, sb1, sW2, sb2):
  br = 1000

  def body(scale_ref, h_ref, r0_ref, r1_ref, d0_ref, d1_ref, w2_ref, b2_ref,
           w1_ref, b1_ref, sw2_ref, sb2_ref, o_ref):
    rsum = r0_ref[...] + r1_ref[...]
    deg = d0_ref[...] + d1_ref[...]
    x = (scale_ref[0] * h_ref[...]
         + jnp.dot(rsum, w2_ref[...], preferred_element_type=jnp.float32)
         + deg * b2_ref[...])
    h2 = jnp.maximum(
        jnp.dot(x, w1_ref[...], preferred_element_type=jnp.float32)
        + b1_ref[...], 0.0)
    o_ref[...] = (
        jnp.dot(h2, sw2_ref[...], preferred_element_type=jnp.float32)
        + sb2_ref[...])

  return pl.pallas_call(
      body,
      grid=(_NN // br,),
      in_specs=[
          pl.BlockSpec(memory_space=pltpu.SMEM),
          pl.BlockSpec((br, _DIM), lambda i: (i, 0)),
          pl.BlockSpec((br, _HID), lambda i: (i, 0)),
          pl.BlockSpec((br, _HID), lambda i: (i, 0)),
          pl.BlockSpec((br, 1), lambda i: (i, 0)),
          pl.BlockSpec((br, 1), lambda i: (i, 0)),
          pl.BlockSpec((_HID, _DIM), lambda i: (0, 0)),
          pl.BlockSpec((1, _DIM), lambda i: (0, 0)),
          pl.BlockSpec((_DIM, _HID), lambda i: (0, 0)),
          pl.BlockSpec((1, _HID), lambda i: (0, 0)),
          pl.BlockSpec((_HID, _DIM), lambda i: (0, 0)),
          pl.BlockSpec((1, _DIM), lambda i: (0, 0)),
      ],
      out_specs=pl.BlockSpec((br, _DIM), lambda i: (i, 0)),
      out_shape=jax.ShapeDtypeStruct((_NN, _DIM), jnp.float32),
  )(scale, H, r0, r1, d0, d1, W2, b2, sW1, sb1, sW2, sb2)


def kernel(H, edge_index, edge_attr, eps, msg_W1, msg_b1, msg_W2, msg_b2,
           self_W1, self_b1, self_W2, self_b2):
  src = edge_index[0].astype(jnp.int32)
  dst = edge_index[1].astype(jnp.int32)
  W1h = msg_W1[:_DIM]
  W1e = msg_W1[_DIM:]

  P = _tc_node_mm(H, W1h, msg_b1.reshape(1, _HID))
  E1 = _tc_e1(edge_attr.T, W1e)
  R, deg = _sc_fused(P, E1, src, dst,
                     jnp.zeros((_NN, _DIM), jnp.float32),
                     jnp.zeros((_NN,), jnp.float32))
  R = R.reshape(_NC, _NN, _HID)
  deg = deg.reshape(_NC, _NN, 1)
  scale = (1.0 + eps).astype(jnp.float32)
  return _tc_final(H, R[0], R[1], deg[0], deg[1], scale, msg_W2,
                   msg_b2.reshape(1, _DIM), self_W1, self_b1.reshape(1, _HID),
                   self_W2, self_b2.reshape(1, _DIM))
